# Initial kernel scaffold; baseline (speedup 1.0000x reference)
import jax
import jax.numpy as jnp
from jax.experimental import pallas as pl


def kernel(uv_out, uv_label):
    raise NotImplementedError("write your pallas kernel here")


# speedup vs baseline: 18.1897x; 18.1897x over previous
import functools

import jax
import jax.numpy as jnp
from jax import lax
from jax.experimental import pallas as pl
from jax.experimental.pallas import tpu as pltpu
from jax.experimental.pallas import tpu_sc as plsc

B, C, N = 4, 20, 65536
K = 4096
CH = 4096
NV = CH // 16
NCHUNK = N // CH
NKV = K // 16
NPAIR = B * C
NC, NS = 2, 16
NW = NC * NS
JMAX = (NPAIR + NW - 1) // NW


def _sc_body(probs_hbm, labels_hbm, res_hbm, pbuf, lbuf, hn, hs, hf, stage):
    wid = lax.axis_index("s") * NC + lax.axis_index("c")
    zeros16 = jnp.zeros((16,), jnp.float32)
    ones16 = jnp.ones((16,), jnp.float32)

    def zero_body(i, _):
        hn[pl.ds(i * 16, 16)] = zeros16
        hs[pl.ds(i * 16, 16)] = zeros16
        hf[pl.ds(i * 16, 16)] = zeros16
        return 0

    lax.fori_loop(0, NKV, zero_body, 0)

    for j in range(JMAX):
        p = wid + NW * j

        @pl.when(p < NPAIR)
        def _():
            b = p // C
            c = p % C

            def chunk_body(ci, _):
                off = ci * CH
                pltpu.sync_copy(probs_hbm.at[b, c, pl.ds(off, CH)], pbuf)
                pltpu.sync_copy(labels_hbm.at[b, pl.ds(off, CH)], lbuf)

                def vec_body(vi, _):
                    o = vi * 16
                    pv = pbuf[pl.ds(o, 16)]
                    lv = lbuf[pl.ds(o, 16)]
                    valid = lv != 0
                    isfg = valid & (lv == c)
                    fgf = jnp.where(isfg, 1.0, 0.0).astype(jnp.float32)
                    e = jnp.where(isfg, 1.0 - pv, pv)
                    bin_ = jnp.minimum((e * K).astype(jnp.int32), K - 1)
                    plsc.addupdate_scatter(hn, [bin_], ones16, mask=valid)
                    plsc.addupdate_scatter(hs, [bin_], e, mask=valid)
                    plsc.addupdate_scatter(hf, [bin_], fgf, mask=valid)
                    return 0

                lax.fori_loop(0, NV, vec_body, 0)
                return 0

            lax.fori_loop(0, NCHUNK, chunk_body, 0)

            def psum_body(i, acc):
                return acc + hf[pl.ds(i * 16, 16)]

            P = jnp.sum(lax.fori_loop(0, NKV, psum_body, zeros16))

            def scan_body(i, carry):
                Tc, Fc, acc = carry
                kc = NKV - 1 - i
                nv = hn[pl.ds(kc * 16, 16)]
                sv = hs[pl.ds(kc * 16, 16)]
                fv = hf[pl.ds(kc * 16, 16)]
                hn[pl.ds(kc * 16, 16)] = zeros16
                hs[pl.ds(kc * 16, 16)] = zeros16
                hf[pl.ds(kc * 16, 16)] = zeros16
                Tin = jnp.flip(jnp.cumsum(jnp.flip(nv, 0)), 0) + Tc
                Fin = jnp.flip(jnp.cumsum(jnp.flip(fv, 0)), 0) + Fc
                Tex = Tin - nv
                Fex = Fin - fv
                Jin = 1.0 - (P - Fin) / jnp.maximum(P + Tin - Fin, 1.0)
                Jex = 1.0 - (P - Fex) / jnp.maximum(P + Tex - Fex, 1.0)
                vbar = sv / jnp.maximum(nv, 1.0)
                acc = acc + vbar * (Jin - Jex)
                return (Tc + jnp.sum(nv), Fc + jnp.sum(fv), acc)

            z = jnp.float32(0.0)
            _, _, acc = lax.fori_loop(0, NKV, scan_body, (z, z, zeros16))
            total_c = jnp.sum(acc)

            lane = lax.iota(jnp.int32, 16)
            row = jnp.where(lane == 0, total_c, jnp.where(lane == 1, P, 0.0))
            stage[...] = row.astype(jnp.float32)
            pltpu.sync_copy(stage, res_hbm.at[p])


_sc_kernel = functools.partial(
    pl.kernel,
    mesh=plsc.VectorSubcoreMesh(core_axis_name="c", subcore_axis_name="s"),
    compiler_params=pltpu.CompilerParams(needs_layout_passes=False),
    out_type=jax.ShapeDtypeStruct((NPAIR, 16), jnp.float32),
    scratch_types=[
        pltpu.VMEM((CH,), jnp.float32),
        pltpu.VMEM((CH,), jnp.int32),
        pltpu.VMEM((K,), jnp.float32),
        pltpu.VMEM((K,), jnp.float32),
        pltpu.VMEM((K,), jnp.float32),
        pltpu.VMEM((16,), jnp.float32),
    ],
)(_sc_body)


def _combine_body(tot_ref, p_ref, out_ref):
    totals = tot_ref[...]
    present = (p_ref[...] > 0).astype(jnp.float32)
    n = jnp.sum(present, axis=1, keepdims=True)
    tsum = jnp.sum(present * totals, axis=1, keepdims=True)
    loss_b = jnp.where(n > 0, tsum / jnp.maximum(n, 1.0), 0.0)
    out_ref[...] = jnp.sum(loss_b, keepdims=True) * (1.0 / B)


def kernel(uv_out, uv_label):
    res = _sc_kernel(uv_out, uv_label)
    totals = res[:, 0].reshape(B, C)
    ps = res[:, 1].reshape(B, C)
    out = pl.pallas_call(
        _combine_body,
        out_shape=jax.ShapeDtypeStruct((1, 1), jnp.float32),
    )(totals, ps)
    return out[0, 0]


# speedup vs baseline: 23.0047x; 1.2647x over previous
import functools

import jax
import jax.numpy as jnp
from jax import lax
from jax.experimental import pallas as pl
from jax.experimental.pallas import tpu as pltpu
from jax.experimental.pallas import tpu_sc as plsc

B, C, N = 4, 20, 65536
K = 2048
CH = 8192
NV = CH // 16
NCHUNK = N // CH
UNROLL = 4
NKV = K // 16
NPAIR = B * C
NC, NS = 2, 16
NW = NC * NS
JMAX = (NPAIR + NW - 1) // NW


def _sc_body(probs_hbm, labels_hbm, res_hbm,
             pbuf0, pbuf1, lbuf0, lbuf1, hn, hs, hf, stage, sem0, sem1):
    wid = lax.axis_index("s") * NC + lax.axis_index("c")
    zeros16 = jnp.zeros((16,), jnp.float32)
    ones16 = jnp.ones((16,), jnp.float32)
    pbufs = (pbuf0, pbuf1)
    lbufs = (lbuf0, lbuf1)
    sems = (sem0, sem1)

    def zero_body(i, _):
        hn[pl.ds(i * 16, 16)] = zeros16
        hs[pl.ds(i * 16, 16)] = zeros16
        hf[pl.ds(i * 16, 16)] = zeros16
        return 0

    lax.fori_loop(0, NKV, zero_body, 0)

    for j in range(JMAX):
        p = wid + NW * j

        @pl.when(p < NPAIR)
        def _():
            b = p // C
            c = p % C

            def start_load(ci, slot):
                off = ci * CH
                pltpu.async_copy(probs_hbm.at[b, c, pl.ds(off, CH)],
                                 pbufs[slot], sems[slot])
                pltpu.async_copy(labels_hbm.at[b, pl.ds(off, CH)],
                                 lbufs[slot], sems[slot])

            def drain(slot):
                pltpu.make_async_copy(probs_hbm.at[0, 0, pl.ds(0, CH)],
                                      pbufs[slot], sems[slot]).wait()
                pltpu.make_async_copy(labels_hbm.at[0, pl.ds(0, CH)],
                                      lbufs[slot], sems[slot]).wait()

            start_load(0, 0)
            acc_fg = zeros16
            for ci in range(NCHUNK):
                slot = ci % 2
                drain(slot)
                if ci + 1 < NCHUNK:
                    start_load(ci + 1, 1 - slot)
                pbuf = pbufs[slot]
                lbuf = lbufs[slot]

                def vec_body(vi, acc):
                    base = vi * (16 * UNROLL)
                    for u in range(UNROLL):
                        o = base + u * 16
                        pv = pbuf[pl.ds(o, 16)]
                        lv = lbuf[pl.ds(o, 16)]
                        valid = lv != 0
                        isfg = valid & (lv == c)
                        fgf = jnp.where(isfg, 1.0, 0.0).astype(jnp.float32)
                        e = jnp.where(isfg, 1.0 - pv, pv)
                        bin_ = jnp.minimum((e * K).astype(jnp.int32), K - 1)
                        plsc.addupdate_scatter(hn, [bin_], ones16, mask=valid)
                        plsc.addupdate_scatter(hs, [bin_], e, mask=valid)
                        plsc.addupdate_scatter(hf, [bin_], fgf, mask=valid)
                        acc = acc + fgf
                    return acc

                acc_fg = lax.fori_loop(0, NV // UNROLL, vec_body, acc_fg)

            P = jnp.sum(acc_fg)

            def scan_body(i, carry):
                Tc, Fc, acc = carry
                kc = NKV - 1 - i
                nv = hn[pl.ds(kc * 16, 16)]
                sv = hs[pl.ds(kc * 16, 16)]
                fv = hf[pl.ds(kc * 16, 16)]
                hn[pl.ds(kc * 16, 16)] = zeros16
                hs[pl.ds(kc * 16, 16)] = zeros16
                hf[pl.ds(kc * 16, 16)] = zeros16
                Tin = jnp.flip(jnp.cumsum(jnp.flip(nv, 0)), 0) + Tc
                Fin = jnp.flip(jnp.cumsum(jnp.flip(fv, 0)), 0) + Fc
                Tex = Tin - nv
                Fex = Fin - fv
                Jin = 1.0 - (P - Fin) / jnp.maximum(P + Tin - Fin, 1.0)
                Jex = 1.0 - (P - Fex) / jnp.maximum(P + Tex - Fex, 1.0)
                vbar = sv / jnp.maximum(nv, 1.0)
                acc = acc + vbar * (Jin - Jex)
                return (Tc + jnp.sum(nv), Fc + jnp.sum(fv), acc)

            z = jnp.float32(0.0)
            _, _, acc = lax.fori_loop(0, NKV, scan_body, (z, z, zeros16))
            total_c = jnp.sum(acc)

            lane = lax.iota(jnp.int32, 16)
            row = jnp.where(lane == 0, total_c, jnp.where(lane == 1, P, 0.0))
            stage[...] = row.astype(jnp.float32)
            pltpu.sync_copy(stage, res_hbm.at[p])


_sc_kernel = functools.partial(
    pl.kernel,
    mesh=plsc.VectorSubcoreMesh(core_axis_name="c", subcore_axis_name="s"),
    compiler_params=pltpu.CompilerParams(needs_layout_passes=False),
    out_type=jax.ShapeDtypeStruct((NPAIR, 16), jnp.float32),
    scratch_types=[
        pltpu.VMEM((CH,), jnp.float32),
        pltpu.VMEM((CH,), jnp.float32),
        pltpu.VMEM((CH,), jnp.int32),
        pltpu.VMEM((CH,), jnp.int32),
        pltpu.VMEM((K,), jnp.float32),
        pltpu.VMEM((K,), jnp.float32),
        pltpu.VMEM((K,), jnp.float32),
        pltpu.VMEM((16,), jnp.float32),
        pltpu.SemaphoreType.DMA,
        pltpu.SemaphoreType.DMA,
    ],
)(_sc_body)


def _combine_body(tot_ref, p_ref, out_ref):
    totals = tot_ref[...]
    present = (p_ref[...] > 0).astype(jnp.float32)
    n = jnp.sum(present, axis=1, keepdims=True)
    tsum = jnp.sum(present * totals, axis=1, keepdims=True)
    loss_b = jnp.where(n > 0, tsum / jnp.maximum(n, 1.0), 0.0)
    out_ref[...] = jnp.sum(loss_b, keepdims=True) * (1.0 / B)


def kernel(uv_out, uv_label):
    res = _sc_kernel(uv_out, uv_label)
    totals = res[:, 0].reshape(B, C)
    ps = res[:, 1].reshape(B, C)
    out = pl.pallas_call(
        _combine_body,
        out_shape=jax.ShapeDtypeStruct((1, 1), jnp.float32),
    )(totals, ps)
    return out[0, 0]


# speedup vs baseline: 56.0468x; 2.4363x over previous
import functools

import jax
import jax.numpy as jnp
from jax import lax
from jax.experimental import pallas as pl
from jax.experimental.pallas import tpu as pltpu
from jax.experimental.pallas import tpu_sc as plsc

B, C, N = 4, 20, 65536
K = 2048
CH = 8192
NV = CH // 16
NCHUNK = N // CH
UNROLL = 8
NKV = K // 16
NPAIR = B * C
NC, NS = 2, 16
NW = NC * NS
JMAX = (NPAIR + NW - 1) // NW


def _splat0(x):
    idx = jnp.zeros((16, 1), jnp.int32)
    return lax.gather(
        x, idx,
        lax.GatherDimensionNumbers(
            offset_dims=(), collapsed_slice_dims=(0,), start_index_map=(0,)),
        (1,), mode=lax.GatherScatterMode.PROMISE_IN_BOUNDS)


def _sc_body(probs_hbm, labels_hbm, res_hbm,
             pbuf0, pbuf1, lbuf0, lbuf1, hn, hs, hf, stage, sem0, sem1):
    wid = lax.axis_index("s") * NC + lax.axis_index("c")
    zeros16 = jnp.zeros((16,), jnp.float32)
    ones16 = jnp.ones((16,), jnp.float32)
    pbufs = (pbuf0, pbuf1)
    lbufs = (lbuf0, lbuf1)
    sems = (sem0, sem1)

    def zero_body(i, _):
        hn[pl.ds(i * 16, 16)] = zeros16
        hs[pl.ds(i * 16, 16)] = zeros16
        hf[pl.ds(i * 16, 16)] = zeros16
        return 0

    lax.fori_loop(0, NKV, zero_body, 0)

    for j in range(JMAX):
        p = wid + NW * j

        @pl.when(p < NPAIR)
        def _():
            b = p // C
            c = p % C
            c0 = jnp.where(c == 0, jnp.int32(-1), c)

            def start_load(ci, slot):
                off = ci * CH
                pltpu.async_copy(probs_hbm.at[b, c, pl.ds(off, CH)],
                                 pbufs[slot], sems[slot])
                pltpu.async_copy(labels_hbm.at[b, pl.ds(off, CH)],
                                 lbufs[slot], sems[slot])

            def drain(slot):
                pltpu.make_async_copy(probs_hbm.at[0, 0, pl.ds(0, CH)],
                                      pbufs[slot], sems[slot]).wait()
                pltpu.make_async_copy(labels_hbm.at[0, pl.ds(0, CH)],
                                      lbufs[slot], sems[slot]).wait()

            start_load(0, 0)
            for ci in range(NCHUNK):
                slot = ci % 2
                drain(slot)
                if ci + 1 < NCHUNK:
                    start_load(ci + 1, 1 - slot)
                pbuf = pbufs[slot]
                lbuf = lbufs[slot]

                def vec_body(vi, _):
                    base = vi * (16 * UNROLL)
                    loaded = []
                    for u in range(UNROLL):
                        o = base + u * 16
                        loaded.append((pbuf[pl.ds(o, 16)], lbuf[pl.ds(o, 16)]))
                    outs = []
                    for pv, lv in loaded:
                        valid = lv != 0
                        isfg = lv == c0
                        e = jnp.where(isfg, 1.0 - pv, pv)
                        ec = jnp.minimum(e, 0.99999994)
                        bin_ = (ec * K).astype(jnp.int32)
                        outs.append((valid, isfg, e, bin_))
                    for valid, isfg, e, bin_ in outs:
                        plsc.addupdate_scatter(hn, [bin_], ones16, mask=valid)
                        plsc.addupdate_scatter(hs, [bin_], e, mask=valid)
                        plsc.addupdate_scatter(hf, [bin_], ones16, mask=isfg)
                    return 0

                lax.fori_loop(0, NV // UNROLL, vec_body, 0)

            def psum_body(i, acc):
                return acc + hf[pl.ds(i * 16, 16)]

            P = jnp.sum(lax.fori_loop(0, NKV, psum_body, zeros16))

            def scan_body(i, carry):
                Tc, Fc, acc = carry
                kc = NKV - 1 - i
                nv = hn[pl.ds(kc * 16, 16)]
                sv = hs[pl.ds(kc * 16, 16)]
                fv = hf[pl.ds(kc * 16, 16)]
                hn[pl.ds(kc * 16, 16)] = zeros16
                hs[pl.ds(kc * 16, 16)] = zeros16
                hf[pl.ds(kc * 16, 16)] = zeros16
                Tin = jnp.flip(jnp.cumsum(jnp.flip(nv, 0)), 0) + Tc
                Fin = jnp.flip(jnp.cumsum(jnp.flip(fv, 0)), 0) + Fc
                Tex = Tin - nv
                Fex = Fin - fv
                Jin = 1.0 - (P - Fin) / jnp.maximum(P + Tin - Fin, 1.0)
                Jex = 1.0 - (P - Fex) / jnp.maximum(P + Tex - Fex, 1.0)
                vbar = sv / jnp.maximum(nv, 1.0)
                acc = acc + vbar * (Jin - Jex)
                return (_splat0(Tin), _splat0(Fin), acc)

            _, _, acc = lax.fori_loop(0, NKV, scan_body,
                                      (zeros16, zeros16, zeros16))
            total_c = jnp.sum(acc)

            lane = lax.iota(jnp.int32, 16)
            row = jnp.where(lane == 0, total_c, jnp.where(lane == 1, P, 0.0))
            stage[...] = row.astype(jnp.float32)
            pltpu.sync_copy(stage, res_hbm.at[p])


_sc_kernel = functools.partial(
    pl.kernel,
    mesh=plsc.VectorSubcoreMesh(core_axis_name="c", subcore_axis_name="s"),
    compiler_params=pltpu.CompilerParams(needs_layout_passes=False),
    out_type=jax.ShapeDtypeStruct((NPAIR, 16), jnp.float32),
    scratch_types=[
        pltpu.VMEM((CH,), jnp.float32),
        pltpu.VMEM((CH,), jnp.float32),
        pltpu.VMEM((CH,), jnp.int32),
        pltpu.VMEM((CH,), jnp.int32),
        pltpu.VMEM((K,), jnp.float32),
        pltpu.VMEM((K,), jnp.float32),
        pltpu.VMEM((K,), jnp.float32),
        pltpu.VMEM((16,), jnp.float32),
        pltpu.SemaphoreType.DMA,
        pltpu.SemaphoreType.DMA,
    ],
)(_sc_body)


def _combine_body(tot_ref, p_ref, out_ref):
    totals = tot_ref[...]
    present = (p_ref[...] > 0).astype(jnp.float32)
    n = jnp.sum(present, axis=1, keepdims=True)
    tsum = jnp.sum(present * totals, axis=1, keepdims=True)
    loss_b = jnp.where(n > 0, tsum / jnp.maximum(n, 1.0), 0.0)
    out_ref[...] = jnp.sum(loss_b, keepdims=True) * (1.0 / B)


def kernel(uv_out, uv_label):
    res = _sc_kernel(uv_out, uv_label)
    totals = res[:, 0].reshape(B, C)
    ps = res[:, 1].reshape(B, C)
    out = pl.pallas_call(
        _combine_body,
        out_shape=jax.ShapeDtypeStruct((1, 1), jnp.float32),
    )(totals, ps)
    return out[0, 0]
